# Initial kernel scaffold; baseline (speedup 1.0000x reference)
import jax
import jax.numpy as jnp
from jax.experimental import pallas as pl


def kernel(added_points, original_points, boxes):
    raise NotImplementedError("write your pallas kernel here")


# speedup vs baseline: 1.6658x; 1.6658x over previous
import jax
import jax.numpy as jnp
from jax import lax
from jax.experimental import pallas as pl


_HB = 64


def _body(added_ref, orig_ref, boxes_ref, out_ref):
    h = pl.program_id(1)
    HB = added_ref.shape[2]
    W = added_ref.shape[3]

    pred = jnp.sum(added_ref[0], axis=0)
    orig = jnp.sum(orig_ref[0, 1:], axis=0)
    occ_p = jnp.abs(pred) > 0.0
    occ_o = jnp.abs(orig) > 0.0
    occ_and = jnp.logical_and(occ_p, occ_o)
    occ_or = jnp.logical_or(occ_p, occ_o)

    row = lax.broadcasted_iota(jnp.int32, (HB, W), 0) + h * HB
    col = lax.broadcasted_iota(jnp.int32, (HB, W), 1)
    x = row.astype(jnp.float32) * 0.8
    y = col.astype(jnp.float32) * 0.8

    bx = boxes_ref[0]
    M = bx.shape[0]
    c = jnp.cos(bx[:, 6:7])
    s = jnp.sin(bx[:, 6:7])
    in_z = jnp.abs(bx[:, 2:3]) < bx[:, 5:6] * 0.5
    ex = jnp.where(in_z, bx[:, 3:4] * 0.5, -1.0)
    ey = bx[:, 4:5] * 0.5

    in_any = jnp.zeros((HB, W), dtype=jnp.bool_)
    for m in range(M):
        sx = x - bx[m : m + 1, 0:1]
        sy = y - bx[m : m + 1, 1:2]
        cm = c[m : m + 1, 0:1]
        sm = s[m : m + 1, 0:1]
        lx = sx * cm + sy * sm
        ly = sy * cm - sx * sm
        hit = jnp.logical_and(
            jnp.abs(lx) < ex[m : m + 1, 0:1],
            jnp.abs(ly) < ey[m : m + 1, 0:1],
        )
        in_any = jnp.logical_or(in_any, hit)

    inter = jnp.sum(jnp.where(jnp.logical_and(in_any, occ_and), 1.0, 0.0))
    union = jnp.sum(jnp.where(jnp.logical_and(in_any, occ_or), 1.0, 0.0))

    lane = lax.broadcasted_iota(jnp.int32, (1, 1, 128), 2)
    v = jnp.where(lane == 0, inter, 0.0) + jnp.where(lane == 1, union, 0.0)

    @pl.when(h == 0)
    def _():
        out_ref[...] = v

    @pl.when(h != 0)
    def _():
        out_ref[...] += v


def kernel(added_points, original_points, boxes):
    B, C, H, W = added_points.shape
    M = boxes.shape[1]
    nh = H // _HB

    out = pl.pallas_call(
        _body,
        grid=(B, nh),
        in_specs=[
            pl.BlockSpec((1, C, _HB, W), lambda b, h: (b, 0, h, 0)),
            pl.BlockSpec((1, C + 1, _HB, W), lambda b, h: (b, 0, h, 0)),
            pl.BlockSpec((1, M, 7), lambda b, h: (b, 0, 0)),
        ],
        out_specs=pl.BlockSpec((1, 1, 128), lambda b, h: (b, 0, 0)),
        out_shape=jax.ShapeDtypeStruct((B, 1, 128), jnp.float32),
    )(added_points, original_points, boxes)

    inter = out[:, 0, 0]
    union = out[:, 0, 1]
    return jnp.mean(M * inter / (union + 1e-6))
